# Initial kernel scaffold; baseline (speedup 1.0000x reference)
import jax
import jax.numpy as jnp
from jax.experimental import pallas as pl


def kernel(x, edge_index, batch, W_enc, b_enc, Wc, bc, gamma, beta, W1, b1, W2, b2):
    raise NotImplementedError("write your pallas kernel here")


# speedup vs baseline: 10.3396x; 10.3396x over previous
import functools

import jax
import jax.numpy as jnp
from jax import lax
from jax.experimental import pallas as pl
from jax.experimental.pallas import tpu as pltpu
from jax.experimental.pallas import tpu_sc as plsc

_NC = 2
_NS = 16
_K = 80
_BR = 1000


def _stripes(n):
    full = -(-(n // _NS) // 8) * 8
    last = n - (_NS - 1) * full
    return full, last


def _striped_rows(s, n, copy_fn):
    full, last = _stripes(n)

    @pl.when(s < _NS - 1)
    def _():
        copy_fn(s * full, full)

    @pl.when(s == _NS - 1)
    def _():
        copy_fn((_NS - 1) * full, last)


def _deg_body(dst_hbm, ones_hbm, zeros_hbm, out_hbm, dst_v, ones_v, acc_sh, sem):
    del sem
    c = lax.axis_index("c")
    s = lax.axis_index("s")
    n = zeros_hbm.shape[0]
    e = dst_hbm.shape[0]
    _striped_rows(s, n, lambda r0, nr: pltpu.sync_copy(
        zeros_hbm.at[pl.ds(r0, nr)], acc_sh.at[pl.ds(r0, nr)]))
    pltpu.sync_copy(ones_hbm, ones_v)
    plsc.subcore_barrier()
    ept = e // (_NC * _NS)
    base = (s * _NC + c) * ept
    nch = ept // _K

    def body(i, carry):
        pltpu.sync_copy(dst_hbm.at[pl.ds(base + i * _K, _K)], dst_v)
        pltpu.sync_copy(ones_v, acc_sh.at[dst_v], add=True)
        return carry

    lax.fori_loop(0, nch, body, 0)
    plsc.subcore_barrier()
    _striped_rows(s, n, lambda r0, nr: pltpu.sync_copy(
        acc_sh.at[pl.ds(r0, nr)], out_hbm.at[pl.ds(c * n + r0, nr)]))


def _msg_body(hn_hbm, src_hbm, dst_hbm, zeros_hbm, out_hbm,
              src_v, dst_v, rows_v, acc_sh, sem):
    c = lax.axis_index("c")
    s = lax.axis_index("s")
    n = zeros_hbm.shape[0]
    e = src_hbm.shape[0]
    _striped_rows(s, n, lambda r0, nr: pltpu.sync_copy(
        zeros_hbm.at[pl.ds(r0, nr)], acc_sh.at[pl.ds(r0, nr)]))
    plsc.subcore_barrier()
    ept = e // (_NC * _NS)
    base = (s * _NC + c) * ept
    nch = ept // _K

    def body(i, carry):
        off = base + i * _K
        pltpu.sync_copy(src_hbm.at[pl.ds(off, _K)], src_v)
        pltpu.async_copy(hn_hbm.at[src_v], rows_v, sem).wait()
        pltpu.sync_copy(dst_hbm.at[pl.ds(off, _K)], dst_v)
        pltpu.sync_copy(rows_v, acc_sh.at[dst_v], add=True)
        return carry

    lax.fori_loop(0, nch, body, 0)
    plsc.subcore_barrier()
    _striped_rows(s, n, lambda r0, nr: pltpu.sync_copy(
        acc_sh.at[pl.ds(r0, nr)], out_hbm.at[pl.ds(c * n + r0, nr)]))


def _sc_degree(dst, n):
    mesh = plsc.VectorSubcoreMesh(core_axis_name="c", subcore_axis_name="s")
    fn = pl.kernel(
        _deg_body,
        out_type=jax.ShapeDtypeStruct((_NC * n, 128), jnp.float32),
        mesh=mesh,
        scratch_types=[
            pltpu.VMEM((_K,), jnp.int32),
            pltpu.VMEM((_K, 128), jnp.float32),
            pltpu.VMEM_SHARED((n, 128), jnp.float32),
            pltpu.SemaphoreType.DMA,
        ],
    )
    return fn(dst, jnp.ones((_K, 128), jnp.float32),
              jnp.zeros((n, 128), jnp.float32))


def _sc_message(hn, src, dst, n, d):
    mesh = plsc.VectorSubcoreMesh(core_axis_name="c", subcore_axis_name="s")
    fn = pl.kernel(
        _msg_body,
        out_type=jax.ShapeDtypeStruct((_NC * n, d), jnp.float32),
        mesh=mesh,
        scratch_types=[
            pltpu.VMEM((_K,), jnp.int32),
            pltpu.VMEM((_K,), jnp.int32),
            pltpu.VMEM((_K, d), jnp.float32),
            pltpu.VMEM_SHARED((n, d), jnp.float32),
            pltpu.SemaphoreType.DMA,
        ],
    )
    return fn(hn, src, dst, jnp.zeros((n, d), jnp.float32))


def _prep_kernel(d0_ref, d1_ref, o_ref):
    deg = d0_ref[:, 0:1] + d1_ref[:, 0:1] + 1.0
    r = lax.rsqrt(deg)
    o_ref[...] = jnp.broadcast_to(r, o_ref.shape)


def _enc_kernel(x_ref, w_ref, b_ref, o_ref):
    o_ref[...] = jnp.dot(x_ref[...], w_ref[...],
                         preferred_element_type=jnp.float32) + b_ref[...]


def _mms_kernel(h_ref, w_ref, s_ref, o_ref):
    o_ref[...] = jnp.dot(h_ref[...], w_ref[...],
                         preferred_element_type=jnp.float32) * s_ref[...]


def _post_kernel(p0_ref, p1_ref, hn_ref, s_ref, bc_ref, g_ref, b_ref, o_ref):
    t = (p0_ref[...] + p1_ref[...] + hn_ref[...]) * s_ref[...] + bc_ref[...]
    mu = jnp.mean(t, axis=1, keepdims=True)
    dlt = t - mu
    var = jnp.mean(dlt * dlt, axis=1, keepdims=True)
    y = dlt * lax.rsqrt(var + 1e-5) * g_ref[...] + b_ref[...]
    o_ref[...] = jnp.maximum(y, 0.0)


def _head_kernel(h_ref, w1_ref, b1_ref, w2_ref, b2_ref, o_ref):
    t = jnp.maximum(jnp.dot(h_ref[...], w1_ref[...],
                            preferred_element_type=jnp.float32) + b1_ref[...],
                    0.0)
    o_ref[...] = jnp.dot(t, w2_ref[...],
                         preferred_element_type=jnp.float32) + b2_ref[...]


def _row_spec(d):
    return pl.BlockSpec((_BR, d), lambda i: (i, 0))


def _full_spec(r, c):
    return pl.BlockSpec((r, c), lambda i: (0, 0))


def _tc_prep(deg2, n):
    nb = n // _BR
    return pl.pallas_call(
        _prep_kernel,
        grid=(nb,),
        in_specs=[
            pl.BlockSpec((_BR, 128), lambda i: (i, 0)),
            pl.BlockSpec((_BR, 128), lambda i: (i + nb, 0)),
        ],
        out_specs=_row_spec(128),
        out_shape=jax.ShapeDtypeStruct((n, 128), jnp.float32),
    )(deg2, deg2)


def _tc_enc(x, w, b):
    n, d = x.shape
    return pl.pallas_call(
        _enc_kernel,
        grid=(n // _BR,),
        in_specs=[_row_spec(d), _full_spec(d, d), _full_spec(1, d)],
        out_specs=_row_spec(d),
        out_shape=jax.ShapeDtypeStruct((n, d), jnp.float32),
    )(x, w, b.reshape(1, d))


def _tc_mms(h, w, dinvb):
    n, d = h.shape
    return pl.pallas_call(
        _mms_kernel,
        grid=(n // _BR,),
        in_specs=[_row_spec(d), _full_spec(d, d), _row_spec(d)],
        out_specs=_row_spec(d),
        out_shape=jax.ShapeDtypeStruct((n, d), jnp.float32),
    )(h, w, dinvb)


def _tc_post(p, hn, dinvb, bc, g, b):
    n, d = hn.shape
    nb = n // _BR
    return pl.pallas_call(
        _post_kernel,
        grid=(nb,),
        in_specs=[
            pl.BlockSpec((_BR, d), lambda i: (i, 0)),
            pl.BlockSpec((_BR, d), lambda i: (i + nb, 0)),
            _row_spec(d), _row_spec(d),
            _full_spec(1, d), _full_spec(1, d), _full_spec(1, d),
        ],
        out_specs=_row_spec(d),
        out_shape=jax.ShapeDtypeStruct((n, d), jnp.float32),
    )(p, p, hn, dinvb, bc.reshape(1, d), g.reshape(1, d), b.reshape(1, d))


def _tc_head(h, w1p, b1p, w2p, b2p):
    n, d = h.shape
    return pl.pallas_call(
        _head_kernel,
        grid=(n // _BR,),
        in_specs=[_row_spec(d), _full_spec(d, d), _full_spec(1, d),
                  _full_spec(d, d), _full_spec(1, d)],
        out_specs=_row_spec(d),
        out_shape=jax.ShapeDtypeStruct((n, d), jnp.float32),
    )(h, w1p, b1p, w2p, b2p)


def kernel(x, edge_index, batch, W_enc, b_enc, Wc, bc, gamma, beta,
           W1, b1, W2, b2):
    del batch
    n, d = x.shape
    src = edge_index[0]
    dst = edge_index[1]

    deg2 = _sc_degree(dst, n)
    dinvb = _tc_prep(deg2, n)
    h = _tc_enc(x, W_enc, b_enc)
    num_layers = Wc.shape[0]
    for i in range(num_layers):
        hn = _tc_mms(h, Wc[i], dinvb)
        p = _sc_message(hn, src, dst, n, d)
        h = _tc_post(p, hn, dinvb, bc[i], gamma[i], beta[i])

    dh = W1.shape[1]
    w1p = jnp.pad(W1, ((0, 0), (0, d - dh)))
    b1p = jnp.pad(b1, (0, d - dh)).reshape(1, d)
    w2p = jnp.pad(W2, ((0, d - dh), (0, d - 1)))
    b2p = jnp.broadcast_to(b2.reshape(1, 1), (1, d))
    out = _tc_head(h, w1p, b1p, w2p, b2p)
    return out[:, :1]


# speedup vs baseline: 20.1153x; 1.9455x over previous
import functools

import jax
import jax.numpy as jnp
from jax import lax
from jax.experimental import pallas as pl
from jax.experimental.pallas import tpu as pltpu
from jax.experimental.pallas import tpu_sc as plsc

_NC = 2
_NS = 16
_K = 80
_BR = 1000


def _stripes(n):
    full = -(-(n // _NS) // 8) * 8
    last = n - (_NS - 1) * full
    return full, last


def _striped_rows(s, n, copy_fn):
    full, last = _stripes(n)

    @pl.when(s < _NS - 1)
    def _():
        copy_fn(s * full, full)

    @pl.when(s == _NS - 1)
    def _():
        copy_fn((_NS - 1) * full, last)


def _deg_body(dst_hbm, ones_hbm, zeros_hbm, out_hbm, dst_v, ones_v, acc_sh, sem):
    del sem
    c = lax.axis_index("c")
    s = lax.axis_index("s")
    n = zeros_hbm.shape[0]
    e = dst_hbm.shape[0]
    _striped_rows(s, n, lambda r0, nr: pltpu.sync_copy(
        zeros_hbm.at[pl.ds(r0, nr)], acc_sh.at[pl.ds(r0, nr)]))
    pltpu.sync_copy(ones_hbm, ones_v)
    plsc.subcore_barrier()
    ept = e // (_NC * _NS)
    base = (s * _NC + c) * ept
    nch = ept // _K

    def body(i, carry):
        pltpu.sync_copy(dst_hbm.at[pl.ds(base + i * _K, _K)], dst_v)
        pltpu.sync_copy(ones_v, acc_sh.at[dst_v], add=True)
        return carry

    lax.fori_loop(0, nch, body, 0)
    plsc.subcore_barrier()
    _striped_rows(s, n, lambda r0, nr: pltpu.sync_copy(
        acc_sh.at[pl.ds(r0, nr)], out_hbm.at[pl.ds(c * n + r0, nr)]))


def _msg_body(hn_hbm, src_hbm, dst_hbm, zeros_hbm, out_hbm,
              src_all, dv_a, dv_b, rv_a, rv_b, acc_sh, sem_a, sem_b):
    c = lax.axis_index("c")
    s = lax.axis_index("s")
    n = zeros_hbm.shape[0]
    e = src_hbm.shape[0]
    ept = e // (_NC * _NS)
    base = (s * _NC + c) * ept
    nch = ept // _K

    pltpu.sync_copy(src_hbm.at[pl.ds(base, ept)], src_all)
    _striped_rows(s, n, lambda r0, nr: pltpu.sync_copy(
        zeros_hbm.at[pl.ds(r0, nr)], acc_sh.at[pl.ds(r0, nr)]))
    plsc.subcore_barrier()

    def start(ch, dv, rv, sem):
        pltpu.async_copy(dst_hbm.at[pl.ds(base + ch * _K, _K)], dv, sem)
        pltpu.async_copy(hn_hbm.at[src_all.at[pl.ds(ch * _K, _K)]], rv, sem)

    def finish(dv, rv, sem):
        pltpu.make_async_copy(dst_hbm.at[pl.ds(base, _K)], dv, sem).wait()
        pltpu.make_async_copy(
            hn_hbm.at[src_all.at[pl.ds(0, _K)]], rv, sem).wait()
        pltpu.sync_copy(rv, acc_sh.at[dv], add=True)

    start(0, dv_a, rv_a, sem_a)

    def body(j, carry):
        start(2 * j + 1, dv_b, rv_b, sem_b)
        finish(dv_a, rv_a, sem_a)
        start(2 * j + 2, dv_a, rv_a, sem_a)
        finish(dv_b, rv_b, sem_b)
        return carry

    lax.fori_loop(0, (nch - 1) // 2, body, 0)
    finish(dv_a, rv_a, sem_a)

    plsc.subcore_barrier()
    _striped_rows(s, n, lambda r0, nr: pltpu.sync_copy(
        acc_sh.at[pl.ds(r0, nr)], out_hbm.at[pl.ds(c * n + r0, nr)]))


def _sc_degree(dst, n):
    mesh = plsc.VectorSubcoreMesh(core_axis_name="c", subcore_axis_name="s")
    fn = pl.kernel(
        _deg_body,
        out_type=jax.ShapeDtypeStruct((_NC * n, 128), jnp.float32),
        mesh=mesh,
        scratch_types=[
            pltpu.VMEM((_K,), jnp.int32),
            pltpu.VMEM((_K, 128), jnp.float32),
            pltpu.VMEM_SHARED((n, 128), jnp.float32),
            pltpu.SemaphoreType.DMA,
        ],
    )
    return fn(dst, jnp.ones((_K, 128), jnp.float32),
              jnp.zeros((n, 128), jnp.float32))


def _sc_message(hn, src, dst, n, d):
    mesh = plsc.VectorSubcoreMesh(core_axis_name="c", subcore_axis_name="s")
    e = src.shape[0]
    fn = pl.kernel(
        _msg_body,
        out_type=jax.ShapeDtypeStruct((_NC * n, d), jnp.float32),
        mesh=mesh,
        scratch_types=[
            pltpu.VMEM((e // (_NC * _NS),), jnp.int32),
            pltpu.VMEM((_K,), jnp.int32),
            pltpu.VMEM((_K,), jnp.int32),
            pltpu.VMEM((_K, d), jnp.float32),
            pltpu.VMEM((_K, d), jnp.float32),
            pltpu.VMEM_SHARED((n, d), jnp.float32),
            pltpu.SemaphoreType.DMA,
            pltpu.SemaphoreType.DMA,
        ],
    )
    return fn(hn, src, dst, jnp.zeros((n, d), jnp.float32))


def _prep_kernel(d0_ref, d1_ref, o_ref):
    deg = d0_ref[:, 0:1] + d1_ref[:, 0:1] + 1.0
    r = lax.rsqrt(deg)
    o_ref[...] = jnp.broadcast_to(r, o_ref.shape)


def _enc_kernel(x_ref, w_ref, b_ref, o_ref):
    o_ref[...] = jnp.dot(x_ref[...], w_ref[...],
                         preferred_element_type=jnp.float32) + b_ref[...]


def _mms_kernel(h_ref, w_ref, s_ref, o_ref):
    o_ref[...] = jnp.dot(h_ref[...], w_ref[...],
                         preferred_element_type=jnp.float32) * s_ref[...]


def _post_kernel(p0_ref, p1_ref, hn_ref, s_ref, bc_ref, g_ref, b_ref, o_ref):
    t = (p0_ref[...] + p1_ref[...] + hn_ref[...]) * s_ref[...] + bc_ref[...]
    mu = jnp.mean(t, axis=1, keepdims=True)
    dlt = t - mu
    var = jnp.mean(dlt * dlt, axis=1, keepdims=True)
    y = dlt * lax.rsqrt(var + 1e-5) * g_ref[...] + b_ref[...]
    o_ref[...] = jnp.maximum(y, 0.0)


def _head_kernel(h_ref, w1_ref, b1_ref, w2_ref, b2_ref, o_ref):
    t = jnp.maximum(jnp.dot(h_ref[...], w1_ref[...],
                            preferred_element_type=jnp.float32) + b1_ref[...],
                    0.0)
    o_ref[...] = jnp.dot(t, w2_ref[...],
                         preferred_element_type=jnp.float32) + b2_ref[...]


def _row_spec(d):
    return pl.BlockSpec((_BR, d), lambda i: (i, 0))


def _full_spec(r, c):
    return pl.BlockSpec((r, c), lambda i: (0, 0))


def _tc_prep(deg2, n):
    nb = n // _BR
    return pl.pallas_call(
        _prep_kernel,
        grid=(nb,),
        in_specs=[
            pl.BlockSpec((_BR, 128), lambda i: (i, 0)),
            pl.BlockSpec((_BR, 128), lambda i: (i + nb, 0)),
        ],
        out_specs=_row_spec(128),
        out_shape=jax.ShapeDtypeStruct((n, 128), jnp.float32),
    )(deg2, deg2)


def _tc_enc(x, w, b):
    n, d = x.shape
    return pl.pallas_call(
        _enc_kernel,
        grid=(n // _BR,),
        in_specs=[_row_spec(d), _full_spec(d, d), _full_spec(1, d)],
        out_specs=_row_spec(d),
        out_shape=jax.ShapeDtypeStruct((n, d), jnp.float32),
    )(x, w, b.reshape(1, d))


def _tc_mms(h, w, dinvb):
    n, d = h.shape
    return pl.pallas_call(
        _mms_kernel,
        grid=(n // _BR,),
        in_specs=[_row_spec(d), _full_spec(d, d), _row_spec(d)],
        out_specs=_row_spec(d),
        out_shape=jax.ShapeDtypeStruct((n, d), jnp.float32),
    )(h, w, dinvb)


def _tc_post(p, hn, dinvb, bc, g, b):
    n, d = hn.shape
    nb = n // _BR
    return pl.pallas_call(
        _post_kernel,
        grid=(nb,),
        in_specs=[
            pl.BlockSpec((_BR, d), lambda i: (i, 0)),
            pl.BlockSpec((_BR, d), lambda i: (i + nb, 0)),
            _row_spec(d), _row_spec(d),
            _full_spec(1, d), _full_spec(1, d), _full_spec(1, d),
        ],
        out_specs=_row_spec(d),
        out_shape=jax.ShapeDtypeStruct((n, d), jnp.float32),
    )(p, p, hn, dinvb, bc.reshape(1, d), g.reshape(1, d), b.reshape(1, d))


def _tc_head(h, w1p, b1p, w2p, b2p):
    n, d = h.shape
    return pl.pallas_call(
        _head_kernel,
        grid=(n // _BR,),
        in_specs=[_row_spec(d), _full_spec(d, d), _full_spec(1, d),
                  _full_spec(d, d), _full_spec(1, d)],
        out_specs=_row_spec(d),
        out_shape=jax.ShapeDtypeStruct((n, d), jnp.float32),
    )(h, w1p, b1p, w2p, b2p)


def kernel(x, edge_index, batch, W_enc, b_enc, Wc, bc, gamma, beta,
           W1, b1, W2, b2):
    del batch
    n, d = x.shape
    src = edge_index[0]
    dst = edge_index[1]

    deg2 = _sc_degree(dst, n)
    dinvb = _tc_prep(deg2, n)
    h = _tc_enc(x, W_enc, b_enc)
    num_layers = Wc.shape[0]
    for i in range(num_layers):
        hn = _tc_mms(h, Wc[i], dinvb)
        p = _sc_message(hn, src, dst, n, d)
        h = _tc_post(p, hn, dinvb, bc[i], gamma[i], beta[i])

    dh = W1.shape[1]
    w1p = jnp.pad(W1, ((0, 0), (0, d - dh)))
    b1p = jnp.pad(b1, (0, d - dh)).reshape(1, d)
    w2p = jnp.pad(W2, ((0, d - dh), (0, d - 1)))
    b2p = jnp.broadcast_to(b2.reshape(1, 1), (1, d))
    out = _tc_head(h, w1p, b1p, w2p, b2p)
    return out[:, :1]


# speedup vs baseline: 22.9433x; 1.1406x over previous
import functools

import jax
import jax.numpy as jnp
from jax import lax
from jax.experimental import pallas as pl
from jax.experimental.pallas import tpu as pltpu
from jax.experimental.pallas import tpu_sc as plsc

_NC = 2
_NS = 16
_K = 80
_BR = 1000


def _stripes(n):
    full = -(-(n // _NS) // 8) * 8
    last = n - (_NS - 1) * full
    return full, last


def _striped_rows(s, n, copy_fn):
    full, last = _stripes(n)

    @pl.when(s < _NS - 1)
    def _():
        copy_fn(s * full, full)

    @pl.when(s == _NS - 1)
    def _():
        copy_fn((_NS - 1) * full, last)


def _deg_body(dst_hbm, ones_hbm, zeros_hbm, out_hbm,
              dv_a, dv_b, ones_v, stripe_v, acc_sh, sem_a, sem_b):
    c = lax.axis_index("c")
    s = lax.axis_index("s")
    n = zeros_hbm.shape[0]
    e = dst_hbm.shape[0]
    ept = e // (_NC * _NS)
    base = (s * _NC + c) * ept
    nch = ept // _K
    full, _ = _stripes(n)

    pltpu.sync_copy(ones_hbm, ones_v)

    def init(r0, nr):
        pltpu.sync_copy(zeros_hbm.at[pl.ds(r0, nr)], stripe_v.at[pl.ds(0, nr)])
        pltpu.sync_copy(stripe_v.at[pl.ds(0, nr)], acc_sh.at[pl.ds(r0, nr)])

    _striped_rows(s, n, init)
    plsc.subcore_barrier()

    def start(ch, dv, sem):
        pltpu.async_copy(dst_hbm.at[pl.ds(base + ch * _K, _K)], dv, sem)

    def finish(dv, sem):
        pltpu.make_async_copy(dst_hbm.at[pl.ds(base, _K)], dv, sem).wait()
        pltpu.sync_copy(ones_v, acc_sh.at[dv], add=True)

    start(0, dv_a, sem_a)

    def body(j, carry):
        start(2 * j + 1, dv_b, sem_b)
        finish(dv_a, sem_a)
        start(2 * j + 2, dv_a, sem_a)
        finish(dv_b, sem_b)
        return carry

    lax.fori_loop(0, (nch - 1) // 2, body, 0)
    finish(dv_a, sem_a)

    plsc.subcore_barrier()

    def writeback(r0, nr):
        pltpu.sync_copy(acc_sh.at[pl.ds(r0, nr)], stripe_v.at[pl.ds(0, nr)])
        pltpu.sync_copy(stripe_v.at[pl.ds(0, nr)],
                        out_hbm.at[pl.ds(c * n + r0, nr)])

    _striped_rows(s, n, writeback)


def _msg_body(hn_hbm, src_hbm, dst_hbm, zeros_hbm, out_hbm,
              src_all, dv_a, dv_b, rv_a, rv_b, acc_sh, sem_a, sem_b):
    c = lax.axis_index("c")
    s = lax.axis_index("s")
    n = zeros_hbm.shape[0]
    e = src_hbm.shape[0]
    ept = e // (_NC * _NS)
    base = (s * _NC + c) * ept
    nch = ept // _K

    pltpu.sync_copy(src_hbm.at[pl.ds(base, ept)], src_all)
    _striped_rows(s, n, lambda r0, nr: pltpu.sync_copy(
        zeros_hbm.at[pl.ds(r0, nr)], acc_sh.at[pl.ds(r0, nr)]))
    plsc.subcore_barrier()

    def start(ch, dv, rv, sem):
        pltpu.async_copy(dst_hbm.at[pl.ds(base + ch * _K, _K)], dv, sem)
        pltpu.async_copy(hn_hbm.at[src_all.at[pl.ds(ch * _K, _K)]], rv, sem)

    def finish(dv, rv, sem):
        pltpu.make_async_copy(dst_hbm.at[pl.ds(base, _K)], dv, sem).wait()
        pltpu.make_async_copy(
            hn_hbm.at[src_all.at[pl.ds(0, _K)]], rv, sem).wait()
        pltpu.sync_copy(rv, acc_sh.at[dv], add=True)

    start(0, dv_a, rv_a, sem_a)

    def body(j, carry):
        start(2 * j + 1, dv_b, rv_b, sem_b)
        finish(dv_a, rv_a, sem_a)
        start(2 * j + 2, dv_a, rv_a, sem_a)
        finish(dv_b, rv_b, sem_b)
        return carry

    lax.fori_loop(0, (nch - 1) // 2, body, 0)
    finish(dv_a, rv_a, sem_a)

    plsc.subcore_barrier()
    _striped_rows(s, n, lambda r0, nr: pltpu.sync_copy(
        acc_sh.at[pl.ds(r0, nr)], out_hbm.at[pl.ds(c * n + r0, nr)]))


def _sc_degree(dst, n):
    mesh = plsc.VectorSubcoreMesh(core_axis_name="c", subcore_axis_name="s")
    fn = pl.kernel(
        _deg_body,
        out_type=jax.ShapeDtypeStruct((_NC * n,), jnp.float32),
        mesh=mesh,
        scratch_types=[
            pltpu.VMEM((_K,), jnp.int32),
            pltpu.VMEM((_K,), jnp.int32),
            pltpu.VMEM((_K,), jnp.float32),
            pltpu.VMEM((_stripes(n)[0],), jnp.float32),
            pltpu.VMEM_SHARED((n,), jnp.float32),
            pltpu.SemaphoreType.DMA,
            pltpu.SemaphoreType.DMA,
        ],
    )
    return fn(dst, jnp.ones((_K,), jnp.float32),
              jnp.zeros((n,), jnp.float32))


def _sc_message(hn, src, dst, n, d):
    mesh = plsc.VectorSubcoreMesh(core_axis_name="c", subcore_axis_name="s")
    e = src.shape[0]
    fn = pl.kernel(
        _msg_body,
        out_type=jax.ShapeDtypeStruct((_NC * n, d), jnp.float32),
        mesh=mesh,
        scratch_types=[
            pltpu.VMEM((e // (_NC * _NS),), jnp.int32),
            pltpu.VMEM((_K,), jnp.int32),
            pltpu.VMEM((_K,), jnp.int32),
            pltpu.VMEM((_K, d), jnp.float32),
            pltpu.VMEM((_K, d), jnp.float32),
            pltpu.VMEM_SHARED((n, d), jnp.float32),
            pltpu.SemaphoreType.DMA,
            pltpu.SemaphoreType.DMA,
        ],
    )
    return fn(hn, src, dst, jnp.zeros((n, d), jnp.float32))


def _prep_kernel(d0_ref, d1_ref, o_ref):
    deg = d0_ref[...] + d1_ref[...] + 1.0
    r = lax.rsqrt(deg)
    o_ref[...] = jnp.broadcast_to(r, o_ref.shape)


def _enc_kernel(x_ref, w_ref, b_ref, o_ref):
    o_ref[...] = jnp.dot(x_ref[...], w_ref[...],
                         preferred_element_type=jnp.float32) + b_ref[...]


def _mms_kernel(h_ref, w_ref, s_ref, o_ref):
    o_ref[...] = jnp.dot(h_ref[...], w_ref[...],
                         preferred_element_type=jnp.float32) * s_ref[...]


def _post_kernel(p0_ref, p1_ref, hn_ref, s_ref, bc_ref, g_ref, b_ref, o_ref):
    t = (p0_ref[...] + p1_ref[...] + hn_ref[...]) * s_ref[...] + bc_ref[...]
    mu = jnp.mean(t, axis=1, keepdims=True)
    dlt = t - mu
    var = jnp.mean(dlt * dlt, axis=1, keepdims=True)
    y = dlt * lax.rsqrt(var + 1e-5) * g_ref[...] + b_ref[...]
    o_ref[...] = jnp.maximum(y, 0.0)


def _head_kernel(h_ref, w1_ref, b1_ref, w2_ref, b2_ref, o_ref):
    t = jnp.maximum(jnp.dot(h_ref[...], w1_ref[...],
                            preferred_element_type=jnp.float32) + b1_ref[...],
                    0.0)
    o_ref[...] = jnp.dot(t, w2_ref[...],
                         preferred_element_type=jnp.float32) + b2_ref[...]


def _row_spec(d):
    return pl.BlockSpec((_BR, d), lambda i: (i, 0))


def _full_spec(r, c):
    return pl.BlockSpec((r, c), lambda i: (0, 0))


def _tc_prep(deg2, n):
    nb = n // _BR
    d2 = deg2.reshape(_NC * n, 1)
    return pl.pallas_call(
        _prep_kernel,
        grid=(nb,),
        in_specs=[
            pl.BlockSpec((_BR, 1), lambda i: (i, 0)),
            pl.BlockSpec((_BR, 1), lambda i: (i + nb, 0)),
        ],
        out_specs=_row_spec(128),
        out_shape=jax.ShapeDtypeStruct((n, 128), jnp.float32),
    )(d2, d2)


def _tc_enc(x, w, b):
    n, d = x.shape
    return pl.pallas_call(
        _enc_kernel,
        grid=(n // _BR,),
        in_specs=[_row_spec(d), _full_spec(d, d), _full_spec(1, d)],
        out_specs=_row_spec(d),
        out_shape=jax.ShapeDtypeStruct((n, d), jnp.float32),
    )(x, w, b.reshape(1, d))


def _tc_mms(h, w, dinvb):
    n, d = h.shape
    return pl.pallas_call(
        _mms_kernel,
        grid=(n // _BR,),
        in_specs=[_row_spec(d), _full_spec(d, d), _row_spec(d)],
        out_specs=_row_spec(d),
        out_shape=jax.ShapeDtypeStruct((n, d), jnp.float32),
    )(h, w, dinvb)


def _tc_post(p, hn, dinvb, bc, g, b):
    n, d = hn.shape
    nb = n // _BR
    return pl.pallas_call(
        _post_kernel,
        grid=(nb,),
        in_specs=[
            pl.BlockSpec((_BR, d), lambda i: (i, 0)),
            pl.BlockSpec((_BR, d), lambda i: (i + nb, 0)),
            _row_spec(d), _row_spec(d),
            _full_spec(1, d), _full_spec(1, d), _full_spec(1, d),
        ],
        out_specs=_row_spec(d),
        out_shape=jax.ShapeDtypeStruct((n, d), jnp.float32),
    )(p, p, hn, dinvb, bc.reshape(1, d), g.reshape(1, d), b.reshape(1, d))


def _tc_head(h, w1p, b1p, w2p, b2p):
    n, d = h.shape
    return pl.pallas_call(
        _head_kernel,
        grid=(n // _BR,),
        in_specs=[_row_spec(d), _full_spec(d, d), _full_spec(1, d),
                  _full_spec(d, d), _full_spec(1, d)],
        out_specs=_row_spec(d),
        out_shape=jax.ShapeDtypeStruct((n, d), jnp.float32),
    )(h, w1p, b1p, w2p, b2p)


def kernel(x, edge_index, batch, W_enc, b_enc, Wc, bc, gamma, beta,
           W1, b1, W2, b2):
    del batch
    n, d = x.shape
    src = edge_index[0]
    dst = edge_index[1]

    deg2 = _sc_degree(dst, n)
    dinvb = _tc_prep(deg2, n)
    h = _tc_enc(x, W_enc, b_enc)
    num_layers = Wc.shape[0]
    for i in range(num_layers):
        hn = _tc_mms(h, Wc[i], dinvb)
        p = _sc_message(hn, src, dst, n, d)
        h = _tc_post(p, hn, dinvb, bc[i], gamma[i], beta[i])

    dh = W1.shape[1]
    w1p = jnp.pad(W1, ((0, 0), (0, d - dh)))
    b1p = jnp.pad(b1, (0, d - dh)).reshape(1, d)
    w2p = jnp.pad(W2, ((0, d - dh), (0, d - 1)))
    b2p = jnp.broadcast_to(b2.reshape(1, 1), (1, d))
    out = _tc_head(h, w1p, b1p, w2p, b2p)
    return out[:, :1]


# speedup vs baseline: 24.7817x; 1.0801x over previous
import functools

import jax
import jax.numpy as jnp
from jax import lax
from jax.experimental import pallas as pl
from jax.experimental.pallas import tpu as pltpu
from jax.experimental.pallas import tpu_sc as plsc

_NC = 2
_NS = 16
_K = 80
_BR = 1000


def _stripes(n):
    full = -(-(n // _NS) // 8) * 8
    last = n - (_NS - 1) * full
    return full, last


def _striped_rows(s, n, copy_fn):
    full, last = _stripes(n)

    @pl.when(s < _NS - 1)
    def _():
        copy_fn(s * full, full)

    @pl.when(s == _NS - 1)
    def _():
        copy_fn((_NS - 1) * full, last)


def _deg_body(dst_hbm, ones_hbm, zeros_hbm, out_hbm,
              dv_a, dv_b, ones_v, stripe_v, acc_sh, sem_a, sem_b):
    c = lax.axis_index("c")
    s = lax.axis_index("s")
    n = zeros_hbm.shape[0]
    e = dst_hbm.shape[0]
    ept = e // (_NC * _NS)
    base = (s * _NC + c) * ept
    nch = ept // _K
    full, _ = _stripes(n)

    pltpu.sync_copy(ones_hbm, ones_v)

    def init(r0, nr):
        pltpu.sync_copy(zeros_hbm.at[pl.ds(r0, nr)], stripe_v.at[pl.ds(0, nr)])
        pltpu.sync_copy(stripe_v.at[pl.ds(0, nr)], acc_sh.at[pl.ds(r0, nr)])

    _striped_rows(s, n, init)
    plsc.subcore_barrier()

    def start(ch, dv, sem):
        pltpu.async_copy(dst_hbm.at[pl.ds(base + ch * _K, _K)], dv, sem)

    def finish(dv, sem):
        pltpu.make_async_copy(dst_hbm.at[pl.ds(base, _K)], dv, sem).wait()
        pltpu.sync_copy(ones_v, acc_sh.at[dv], add=True)

    start(0, dv_a, sem_a)

    def body(j, carry):
        start(2 * j + 1, dv_b, sem_b)
        finish(dv_a, sem_a)
        start(2 * j + 2, dv_a, sem_a)
        finish(dv_b, sem_b)
        return carry

    lax.fori_loop(0, (nch - 1) // 2, body, 0)
    finish(dv_a, sem_a)

    plsc.subcore_barrier()

    def writeback(r0, nr):
        pltpu.sync_copy(acc_sh.at[pl.ds(r0, nr)], stripe_v.at[pl.ds(0, nr)])
        pltpu.sync_copy(stripe_v.at[pl.ds(0, nr)],
                        out_hbm.at[pl.ds(c * n + r0, nr)])

    _striped_rows(s, n, writeback)


def _msg_body(hn_hbm, src_hbm, dst_hbm, zeros_hbm, out_hbm,
              src_all, dv_a, dv_b, rv_a, rv_b, acc_sh, sem_a, sem_b):
    c = lax.axis_index("c")
    s = lax.axis_index("s")
    n = zeros_hbm.shape[0]
    e = src_hbm.shape[0]
    ept = e // (_NC * _NS)
    base = (s * _NC + c) * ept
    nch = ept // _K

    pltpu.sync_copy(src_hbm.at[pl.ds(base, ept)], src_all)
    _striped_rows(s, n, lambda r0, nr: pltpu.sync_copy(
        zeros_hbm.at[pl.ds(r0, nr)], acc_sh.at[pl.ds(r0, nr)]))
    plsc.subcore_barrier()

    def start(ch, dv, rv, sem):
        pltpu.async_copy(dst_hbm.at[pl.ds(base + ch * _K, _K)], dv, sem)
        pltpu.async_copy(hn_hbm.at[src_all.at[pl.ds(ch * _K, _K)]], rv, sem)

    def finish(dv, rv, sem):
        pltpu.make_async_copy(dst_hbm.at[pl.ds(base, _K)], dv, sem).wait()
        pltpu.make_async_copy(
            hn_hbm.at[src_all.at[pl.ds(0, _K)]], rv, sem).wait()
        pltpu.sync_copy(rv, acc_sh.at[dv], add=True)

    start(0, dv_a, rv_a, sem_a)

    def body(j, carry):
        start(2 * j + 1, dv_b, rv_b, sem_b)
        finish(dv_a, rv_a, sem_a)
        start(2 * j + 2, dv_a, rv_a, sem_a)
        finish(dv_b, rv_b, sem_b)
        return carry

    lax.fori_loop(0, (nch - 1) // 2, body, 0)
    finish(dv_a, rv_a, sem_a)

    plsc.subcore_barrier()
    _striped_rows(s, n, lambda r0, nr: pltpu.sync_copy(
        acc_sh.at[pl.ds(r0, nr)], out_hbm.at[pl.ds(c * n + r0, nr)]))


def _sc_degree(dst, n):
    mesh = plsc.VectorSubcoreMesh(core_axis_name="c", subcore_axis_name="s")
    fn = pl.kernel(
        _deg_body,
        out_type=jax.ShapeDtypeStruct((_NC * n,), jnp.float32),
        mesh=mesh,
        scratch_types=[
            pltpu.VMEM((_K,), jnp.int32),
            pltpu.VMEM((_K,), jnp.int32),
            pltpu.VMEM((_K,), jnp.float32),
            pltpu.VMEM((_stripes(n)[0],), jnp.float32),
            pltpu.VMEM_SHARED((n,), jnp.float32),
            pltpu.SemaphoreType.DMA,
            pltpu.SemaphoreType.DMA,
        ],
    )
    return fn(dst, jnp.ones((_K,), jnp.float32),
              jnp.zeros((n,), jnp.float32))


def _sc_message(hn, src, dst, n, d):
    mesh = plsc.VectorSubcoreMesh(core_axis_name="c", subcore_axis_name="s")
    e = src.shape[0]
    fn = pl.kernel(
        _msg_body,
        out_type=jax.ShapeDtypeStruct((_NC * n, d), jnp.float32),
        mesh=mesh,
        scratch_types=[
            pltpu.VMEM((e // (_NC * _NS),), jnp.int32),
            pltpu.VMEM((_K,), jnp.int32),
            pltpu.VMEM((_K,), jnp.int32),
            pltpu.VMEM((_K, d), jnp.float32),
            pltpu.VMEM((_K, d), jnp.float32),
            pltpu.VMEM_SHARED((n, d), jnp.float32),
            pltpu.SemaphoreType.DMA,
            pltpu.SemaphoreType.DMA,
        ],
    )
    return fn(hn, src, dst, jnp.zeros((n, d), jnp.float32))


def _dot(a, b):
    return jnp.dot(a, b, preferred_element_type=jnp.float32)


def _encA_kernel(d0_ref, d1_ref, x_ref, we_ref, be_ref, wc_ref,
                 dinv_o, hn_o):
    deg = d0_ref[...] + d1_ref[...] + 1.0
    dinvb = jnp.broadcast_to(lax.rsqrt(deg), dinv_o.shape)
    dinv_o[...] = dinvb
    h = _dot(x_ref[...], we_ref[...]) + be_ref[...]
    hn_o[...] = _dot(h, wc_ref[...]) * dinvb


def _ln_relu(p0, p1, hn, dinv, bc, g, b):
    t = (p0 + p1 + hn) * dinv + bc
    mu = jnp.mean(t, axis=1, keepdims=True)
    dlt = t - mu
    var = jnp.mean(dlt * dlt, axis=1, keepdims=True)
    y = dlt * lax.rsqrt(var + 1e-5) * g + b
    return jnp.maximum(y, 0.0)


def _postB_kernel(p0_ref, p1_ref, hn_ref, s_ref, bc_ref, g_ref, b_ref,
                  wc_ref, hn_o):
    h = _ln_relu(p0_ref[...], p1_ref[...], hn_ref[...], s_ref[...],
                 bc_ref[...], g_ref[...], b_ref[...])
    hn_o[...] = _dot(h, wc_ref[...]) * s_ref[...]


def _postC_kernel(p0_ref, p1_ref, hn_ref, s_ref, bc_ref, g_ref, b_ref,
                  w1_ref, b1_ref, w2_ref, b2_ref, o_ref):
    h = _ln_relu(p0_ref[...], p1_ref[...], hn_ref[...], s_ref[...],
                 bc_ref[...], g_ref[...], b_ref[...])
    t = jnp.maximum(_dot(h, w1_ref[...]) + b1_ref[...], 0.0)
    o_ref[...] = _dot(t, w2_ref[...]) + b2_ref[...]


def _row_spec(d):
    return pl.BlockSpec((_BR, d), lambda i: (i, 0))


def _full_spec(r, c):
    return pl.BlockSpec((r, c), lambda i: (0, 0))


def _tc_encA(deg2, x, we, be, wc):
    n, d = x.shape
    nb = n // _BR
    d2 = deg2.reshape(_NC * n, 1)
    return pl.pallas_call(
        _encA_kernel,
        grid=(nb,),
        in_specs=[
            pl.BlockSpec((_BR, 1), lambda i: (i, 0)),
            pl.BlockSpec((_BR, 1), lambda i: (i + nb, 0)),
            _row_spec(d), _full_spec(d, d), _full_spec(1, d),
            _full_spec(d, d),
        ],
        out_specs=(_row_spec(d), _row_spec(d)),
        out_shape=(jax.ShapeDtypeStruct((n, d), jnp.float32),
                   jax.ShapeDtypeStruct((n, d), jnp.float32)),
    )(d2, d2, x, we, be.reshape(1, d), wc)


def _part_specs(d, nb):
    return [pl.BlockSpec((_BR, d), lambda i: (i, 0)),
            pl.BlockSpec((_BR, d), lambda i: (i + nb, 0))]


def _tc_postB(p, hn, dinvb, bc, g, b, wc):
    n, d = hn.shape
    nb = n // _BR
    return pl.pallas_call(
        _postB_kernel,
        grid=(nb,),
        in_specs=_part_specs(d, nb) + [
            _row_spec(d), _row_spec(d),
            _full_spec(1, d), _full_spec(1, d), _full_spec(1, d),
            _full_spec(d, d),
        ],
        out_specs=_row_spec(d),
        out_shape=jax.ShapeDtypeStruct((n, d), jnp.float32),
    )(p, p, hn, dinvb, bc.reshape(1, d), g.reshape(1, d), b.reshape(1, d), wc)


def _tc_postC(p, hn, dinvb, bc, g, b, w1p, b1p, w2p, b2p):
    n, d = hn.shape
    nb = n // _BR
    return pl.pallas_call(
        _postC_kernel,
        grid=(nb,),
        in_specs=_part_specs(d, nb) + [
            _row_spec(d), _row_spec(d),
            _full_spec(1, d), _full_spec(1, d), _full_spec(1, d),
            _full_spec(d, d), _full_spec(1, d),
            _full_spec(d, d), _full_spec(1, d),
        ],
        out_specs=_row_spec(d),
        out_shape=jax.ShapeDtypeStruct((n, d), jnp.float32),
    )(p, p, hn, dinvb, bc.reshape(1, d), g.reshape(1, d), b.reshape(1, d),
      w1p, b1p, w2p, b2p)


def kernel(x, edge_index, batch, W_enc, b_enc, Wc, bc, gamma, beta,
           W1, b1, W2, b2):
    del batch
    n, d = x.shape
    src = edge_index[0]
    dst = edge_index[1]

    deg2 = _sc_degree(dst, n)
    dinvb, hn = _tc_encA(deg2, x, W_enc, b_enc, Wc[0])

    dh = W1.shape[1]
    w1p = jnp.pad(W1, ((0, 0), (0, d - dh)))
    b1p = jnp.pad(b1, (0, d - dh)).reshape(1, d)
    w2p = jnp.pad(W2, ((0, d - dh), (0, d - 1)))
    b2p = jnp.broadcast_to(b2.reshape(1, 1), (1, d))

    num_layers = Wc.shape[0]
    for i in range(num_layers):
        p = _sc_message(hn, src, dst, n, d)
        if i + 1 < num_layers:
            hn = _tc_postB(p, hn, dinvb, bc[i], gamma[i], beta[i], Wc[i + 1])
        else:
            out = _tc_postC(p, hn, dinvb, bc[i], gamma[i], beta[i],
                            w1p, b1p, w2p, b2p)
    return out[:, :1]


# speedup vs baseline: 27.1965x; 1.0974x over previous
import functools

import jax
import jax.numpy as jnp
from jax import lax
from jax.experimental import pallas as pl
from jax.experimental.pallas import tpu as pltpu
from jax.experimental.pallas import tpu_sc as plsc

_NC = 2
_NS = 16
_K = 128
_BR = 1000


def _stripes(n):
    full = -(-(n // _NS) // 8) * 8
    last = n - (_NS - 1) * full
    return full, last


def _striped_rows(s, n, copy_fn):
    full, last = _stripes(n)

    @pl.when(s < _NS - 1)
    def _():
        copy_fn(s * full, full)

    @pl.when(s == _NS - 1)
    def _():
        copy_fn((_NS - 1) * full, last)


def _deg_body(dst_hbm, ones_hbm, zeros_hbm, out_hbm,
              dv_a, dv_b, dv_t, ones_v, stripe_v, acc_sh, sem_a, sem_b):
    c = lax.axis_index("c")
    s = lax.axis_index("s")
    n = zeros_hbm.shape[0]
    e = dst_hbm.shape[0]
    ept = e // (_NC * _NS)
    base = (s * _NC + c) * ept
    nch = ept // _K
    full, _ = _stripes(n)

    pltpu.sync_copy(ones_hbm, ones_v)

    def init(r0, nr):
        pltpu.sync_copy(zeros_hbm.at[pl.ds(r0, nr)], stripe_v.at[pl.ds(0, nr)])
        pltpu.sync_copy(stripe_v.at[pl.ds(0, nr)], acc_sh.at[pl.ds(r0, nr)])

    _striped_rows(s, n, init)
    plsc.subcore_barrier()

    nbig = ept // _K
    tail = ept - nbig * _K

    def start(ch, dv, sem, sz=_K):
        pltpu.async_copy(dst_hbm.at[pl.ds(base + ch * _K, sz)], dv, sem)

    def finish(dv, sem, ones=ones_v, sz=_K):
        pltpu.make_async_copy(dst_hbm.at[pl.ds(base, sz)], dv, sem).wait()
        pltpu.sync_copy(ones, acc_sh.at[dv], add=True)

    start(0, dv_a, sem_a)

    def body(j, carry):
        start(2 * j + 1, dv_b, sem_b)
        finish(dv_a, sem_a)
        start(2 * j + 2, dv_a, sem_a)
        finish(dv_b, sem_b)
        return carry

    lax.fori_loop(0, (nbig - 2) // 2, body, 0)
    start(nbig - 1, dv_b, sem_b)
    finish(dv_a, sem_a)
    finish(dv_b, sem_b)
    if tail:
        start(nbig, dv_t, sem_a, sz=tail)
        finish(dv_t, sem_a, ones=ones_v.at[pl.ds(0, tail)], sz=tail)

    plsc.subcore_barrier()

    def writeback(r0, nr):
        pltpu.sync_copy(acc_sh.at[pl.ds(r0, nr)], stripe_v.at[pl.ds(0, nr)])
        pltpu.sync_copy(stripe_v.at[pl.ds(0, nr)],
                        out_hbm.at[pl.ds(c * n + r0, nr)])

    _striped_rows(s, n, writeback)


def _msg_body(hn_hbm, src_hbm, dst_hbm, zeros_hbm, out_hbm,
              src_all, dv_a, dv_b, dv_t, rv_a, rv_b, rv_t,
              acc_sh, sem_a, sem_b):
    c = lax.axis_index("c")
    s = lax.axis_index("s")
    n = zeros_hbm.shape[0]
    e = src_hbm.shape[0]
    ept = e // (_NC * _NS)
    base = (s * _NC + c) * ept
    nbig = ept // _K
    tail = ept - nbig * _K

    pltpu.sync_copy(src_hbm.at[pl.ds(base, ept)], src_all)
    _striped_rows(s, n, lambda r0, nr: pltpu.sync_copy(
        zeros_hbm.at[pl.ds(r0, nr)], acc_sh.at[pl.ds(r0, nr)]))
    plsc.subcore_barrier()

    def start(ch, dv, rv, sem, sz=_K):
        pltpu.async_copy(dst_hbm.at[pl.ds(base + ch * _K, sz)], dv, sem)
        pltpu.async_copy(hn_hbm.at[src_all.at[pl.ds(ch * _K, sz)]], rv, sem)

    def finish(dv, rv, sem, sz=_K):
        pltpu.make_async_copy(dst_hbm.at[pl.ds(base, sz)], dv, sem).wait()
        pltpu.make_async_copy(
            hn_hbm.at[src_all.at[pl.ds(0, sz)]], rv, sem).wait()
        pltpu.sync_copy(rv, acc_sh.at[dv], add=True)

    start(0, dv_a, rv_a, sem_a)

    def body(j, carry):
        start(2 * j + 1, dv_b, rv_b, sem_b)
        finish(dv_a, rv_a, sem_a)
        start(2 * j + 2, dv_a, rv_a, sem_a)
        finish(dv_b, rv_b, sem_b)
        return carry

    lax.fori_loop(0, (nbig - 2) // 2, body, 0)
    start(nbig - 1, dv_b, rv_b, sem_b)
    finish(dv_a, rv_a, sem_a)
    finish(dv_b, rv_b, sem_b)
    if tail:
        start(nbig, dv_t, rv_t, sem_a, sz=tail)
        finish(dv_t, rv_t, sem_a, sz=tail)

    plsc.subcore_barrier()
    _striped_rows(s, n, lambda r0, nr: pltpu.sync_copy(
        acc_sh.at[pl.ds(r0, nr)], out_hbm.at[pl.ds(c * n + r0, nr)]))


def _sc_degree(dst, n):
    mesh = plsc.VectorSubcoreMesh(core_axis_name="c", subcore_axis_name="s")
    fn = pl.kernel(
        _deg_body,
        out_type=jax.ShapeDtypeStruct((_NC * n,), jnp.float32),
        mesh=mesh,
        scratch_types=[
            pltpu.VMEM((_K,), jnp.int32),
            pltpu.VMEM((_K,), jnp.int32),
            pltpu.VMEM((dst.shape[0] // (_NC * _NS) % _K or _K,), jnp.int32),
            pltpu.VMEM((_K,), jnp.float32),
            pltpu.VMEM((_stripes(n)[0],), jnp.float32),
            pltpu.VMEM_SHARED((n,), jnp.float32),
            pltpu.SemaphoreType.DMA,
            pltpu.SemaphoreType.DMA,
        ],
    )
    return fn(dst, jnp.ones((_K,), jnp.float32),
              jnp.zeros((n,), jnp.float32))


def _sc_message(hn, src, dst, n, d):
    mesh = plsc.VectorSubcoreMesh(core_axis_name="c", subcore_axis_name="s")
    e = src.shape[0]
    fn = pl.kernel(
        _msg_body,
        out_type=jax.ShapeDtypeStruct((_NC * n, d), jnp.float32),
        mesh=mesh,
        scratch_types=[
            pltpu.VMEM((e // (_NC * _NS),), jnp.int32),
            pltpu.VMEM((_K,), jnp.int32),
            pltpu.VMEM((_K,), jnp.int32),
            pltpu.VMEM((e // (_NC * _NS) % _K or _K,), jnp.int32),
            pltpu.VMEM((_K, d), jnp.float32),
            pltpu.VMEM((_K, d), jnp.float32),
            pltpu.VMEM((e // (_NC * _NS) % _K or _K, d), jnp.float32),
            pltpu.VMEM_SHARED((n, d), jnp.float32),
            pltpu.SemaphoreType.DMA,
            pltpu.SemaphoreType.DMA,
        ],
    )
    return fn(hn, src, dst, jnp.zeros((n, d), jnp.float32))


def _dot(a, b):
    return jnp.dot(a, b, preferred_element_type=jnp.float32)


def _encA_kernel(d0_ref, d1_ref, x_ref, we_ref, be_ref, wc_ref,
                 dinv_o, hn_o):
    deg = d0_ref[...] + d1_ref[...] + 1.0
    dinvb = jnp.broadcast_to(lax.rsqrt(deg), dinv_o.shape)
    dinv_o[...] = dinvb
    h = _dot(x_ref[...], we_ref[...]) + be_ref[...]
    hn_o[...] = _dot(h, wc_ref[...]) * dinvb


def _ln_relu(p0, p1, hn, dinv, bc, g, b):
    t = (p0 + p1 + hn) * dinv + bc
    mu = jnp.mean(t, axis=1, keepdims=True)
    dlt = t - mu
    var = jnp.mean(dlt * dlt, axis=1, keepdims=True)
    y = dlt * lax.rsqrt(var + 1e-5) * g + b
    return jnp.maximum(y, 0.0)


def _postB_kernel(p0_ref, p1_ref, hn_ref, s_ref, bc_ref, g_ref, b_ref,
                  wc_ref, hn_o):
    h = _ln_relu(p0_ref[...], p1_ref[...], hn_ref[...], s_ref[...],
                 bc_ref[...], g_ref[...], b_ref[...])
    hn_o[...] = _dot(h, wc_ref[...]) * s_ref[...]


def _postC_kernel(p0_ref, p1_ref, hn_ref, s_ref, bc_ref, g_ref, b_ref,
                  w1_ref, b1_ref, w2_ref, b2_ref, o_ref):
    h = _ln_relu(p0_ref[...], p1_ref[...], hn_ref[...], s_ref[...],
                 bc_ref[...], g_ref[...], b_ref[...])
    t = jnp.maximum(_dot(h, w1_ref[...]) + b1_ref[...], 0.0)
    o_ref[...] = _dot(t, w2_ref[...]) + b2_ref[...]


def _row_spec(d):
    return pl.BlockSpec((_BR, d), lambda i: (i, 0))


def _full_spec(r, c):
    return pl.BlockSpec((r, c), lambda i: (0, 0))


def _tc_encA(deg2, x, we, be, wc):
    n, d = x.shape
    nb = n // _BR
    d2 = deg2.reshape(_NC * n, 1)
    return pl.pallas_call(
        _encA_kernel,
        grid=(nb,),
        in_specs=[
            pl.BlockSpec((_BR, 1), lambda i: (i, 0)),
            pl.BlockSpec((_BR, 1), lambda i: (i + nb, 0)),
            _row_spec(d), _full_spec(d, d), _full_spec(1, d),
            _full_spec(d, d),
        ],
        out_specs=(_row_spec(d), _row_spec(d)),
        out_shape=(jax.ShapeDtypeStruct((n, d), jnp.float32),
                   jax.ShapeDtypeStruct((n, d), jnp.float32)),
    )(d2, d2, x, we, be.reshape(1, d), wc)


def _part_specs(d, nb):
    return [pl.BlockSpec((_BR, d), lambda i: (i, 0)),
            pl.BlockSpec((_BR, d), lambda i: (i + nb, 0))]


def _tc_postB(p, hn, dinvb, bc, g, b, wc):
    n, d = hn.shape
    nb = n // _BR
    return pl.pallas_call(
        _postB_kernel,
        grid=(nb,),
        in_specs=_part_specs(d, nb) + [
            _row_spec(d), _row_spec(d),
            _full_spec(1, d), _full_spec(1, d), _full_spec(1, d),
            _full_spec(d, d),
        ],
        out_specs=_row_spec(d),
        out_shape=jax.ShapeDtypeStruct((n, d), jnp.float32),
    )(p, p, hn, dinvb, bc.reshape(1, d), g.reshape(1, d), b.reshape(1, d), wc)


def _tc_postC(p, hn, dinvb, bc, g, b, w1p, b1p, w2p, b2p):
    n, d = hn.shape
    nb = n // _BR
    return pl.pallas_call(
        _postC_kernel,
        grid=(nb,),
        in_specs=_part_specs(d, nb) + [
            _row_spec(d), _row_spec(d),
            _full_spec(1, d), _full_spec(1, d), _full_spec(1, d),
            _full_spec(d, d), _full_spec(1, d),
            _full_spec(d, d), _full_spec(1, d),
        ],
        out_specs=_row_spec(d),
        out_shape=jax.ShapeDtypeStruct((n, d), jnp.float32),
    )(p, p, hn, dinvb, bc.reshape(1, d), g.reshape(1, d), b.reshape(1, d),
      w1p, b1p, w2p, b2p)


def kernel(x, edge_index, batch, W_enc, b_enc, Wc, bc, gamma, beta,
           W1, b1, W2, b2):
    del batch
    n, d = x.shape
    src = edge_index[0]
    dst = edge_index[1]

    deg2 = _sc_degree(dst, n)
    dinvb, hn = _tc_encA(deg2, x, W_enc, b_enc, Wc[0])

    dh = W1.shape[1]
    w1p = jnp.pad(W1, ((0, 0), (0, d - dh)))
    b1p = jnp.pad(b1, (0, d - dh)).reshape(1, d)
    w2p = jnp.pad(W2, ((0, d - dh), (0, d - 1)))
    b2p = jnp.broadcast_to(b2.reshape(1, 1), (1, d))

    num_layers = Wc.shape[0]
    for i in range(num_layers):
        p = _sc_message(hn, src, dst, n, d)
        if i + 1 < num_layers:
            hn = _tc_postB(p, hn, dinvb, bc[i], gamma[i], beta[i], Wc[i + 1])
        else:
            out = _tc_postC(p, hn, dinvb, bc[i], gamma[i], beta[i],
                            w1p, b1p, w2p, b2p)
    return out[:, :1]
